# baseline (speedup 1.0000x reference)
import functools

import jax
import jax.numpy as jnp
from jax import lax
from jax.experimental import pallas as pl
from jax.experimental.pallas import tpu as pltpu
from jax.experimental.pallas import tpu_sc as plsc

_EMBED_DIM = 128
_BATCH = 16384

_info = plsc.get_sparse_core_info()
_NC, _NS = _info.num_cores, _info.num_subcores
_NW = _NC * _NS
_B_PER_W = _BATCH // _NW
_CHUNK = 128
_NCH = _B_PER_W // _CHUNK

_mesh = plsc.VectorSubcoreMesh(core_axis_name="c", subcore_axis_name="s")


@functools.partial(
    pl.kernel,
    mesh=_mesh,
    out_type=jax.ShapeDtypeStruct((_BATCH, _EMBED_DIM), jnp.float32),
    scratch_types=[
        pltpu.VMEM((_B_PER_W,), jnp.int32),
        pltpu.VMEM((_B_PER_W, _EMBED_DIM), jnp.float32),
    ]
    + [pltpu.SemaphoreType.DMA] * (_NCH + 1),
)
def _gather_kernel(idx_hbm, table_hbm, out_hbm, idx_v, rows_v, *sems):
    gsems, wsem = sems[:_NCH], sems[_NCH]
    wid = lax.axis_index("s") * _NC + lax.axis_index("c")
    base = wid * _B_PER_W
    pltpu.sync_copy(idx_hbm.at[pl.ds(base, _B_PER_W)], idx_v)
    rds = [
        pltpu.async_copy(
            table_hbm.at[idx_v.at[pl.ds(c * _CHUNK, _CHUNK)]],
            rows_v.at[pl.ds(c * _CHUNK, _CHUNK)],
            gsems[c],
        )
        for c in range(_NCH)
    ]
    wrs = []
    for c in range(_NCH):
        rds[c].wait()
        wrs.append(
            pltpu.async_copy(
                rows_v.at[pl.ds(c * _CHUNK, _CHUNK)],
                out_hbm.at[pl.ds(base + c * _CHUNK, _CHUNK)],
                wsem,
            )
        )
    for w in wrs:
        w.wait()


def kernel(t, embedding_weight):
    return _gather_kernel(t.astype(jnp.int32), embedding_weight)


# speedup vs baseline: 1.1218x; 1.1218x over previous
import functools

import jax
import jax.numpy as jnp
from jax import lax
from jax.experimental import pallas as pl
from jax.experimental.pallas import tpu as pltpu
from jax.experimental.pallas import tpu_sc as plsc

_ROWS = 1000
_EMBED_DIM = 128
_BATCH = 16384

_info = plsc.get_sparse_core_info()
_NC, _NS = _info.num_cores, _info.num_subcores
_NW = _NC * _NS
_B_PER_W = _BATCH // _NW
_ROWS_PER_TILE = 64

_mesh = plsc.VectorSubcoreMesh(core_axis_name="c", subcore_axis_name="s")


@functools.partial(
    pl.kernel,
    mesh=_mesh,
    out_type=jax.ShapeDtypeStruct((_BATCH, _EMBED_DIM), jnp.float32),
    scratch_types=[
        pltpu.VMEM((_B_PER_W,), jnp.int32),
        pltpu.VMEM((_B_PER_W, _EMBED_DIM), jnp.float32),
        pltpu.VMEM_SHARED((_ROWS, _EMBED_DIM), jnp.float32),
        pltpu.SemaphoreType.DMA,
    ],
)
def _gather_kernel(idx_hbm, table_hbm, out_hbm, idx_v, rows_v, table_sh, sem):
    sid = lax.axis_index("s")
    wid = sid * _NC + lax.axis_index("c")
    base = wid * _B_PER_W
    r0 = jnp.minimum(sid * _ROWS_PER_TILE, _ROWS - _ROWS_PER_TILE)
    pltpu.sync_copy(table_hbm.at[pl.ds(r0, _ROWS_PER_TILE)],
                    table_sh.at[pl.ds(r0, _ROWS_PER_TILE)])

    pltpu.sync_copy(idx_hbm.at[pl.ds(base, _B_PER_W)], idx_v)
    plsc.subcore_barrier()
    pltpu.async_copy(table_sh.at[idx_v], rows_v, sem).wait()
    pltpu.sync_copy(rows_v, out_hbm.at[pl.ds(base, _B_PER_W)])


def kernel(t, embedding_weight):
    return _gather_kernel(t.astype(jnp.int32), embedding_weight)


# speedup vs baseline: 1.1517x; 1.0267x over previous
import functools

import jax
import jax.numpy as jnp
from jax import lax
from jax.experimental import pallas as pl
from jax.experimental.pallas import tpu as pltpu
from jax.experimental.pallas import tpu_sc as plsc

_ROWS = 1000
_EMBED_DIM = 128
_BATCH = 16384

_info = plsc.get_sparse_core_info()
_NC, _NS = _info.num_cores, _info.num_subcores
_NW = _NC * _NS
_B_PER_W = _BATCH // _NW
_CHUNK = 128
_NCH = _B_PER_W // _CHUNK
_ROWS_PER_TILE = 64

_mesh = plsc.VectorSubcoreMesh(core_axis_name="c", subcore_axis_name="s")


@functools.partial(
    pl.kernel,
    mesh=_mesh,
    out_type=jax.ShapeDtypeStruct((_BATCH, _EMBED_DIM), jnp.float32),
    scratch_types=[
        pltpu.VMEM((_B_PER_W,), jnp.int32),
        pltpu.VMEM((_B_PER_W, _EMBED_DIM), jnp.float32),
        pltpu.VMEM_SHARED((_ROWS, _EMBED_DIM), jnp.float32),
        pltpu.SemaphoreType.DMA,
        pltpu.SemaphoreType.DMA,
    ],
)
def _gather_kernel(idx_hbm, table_hbm, out_hbm, idx_v, rows_v, table_sh, gsem, wsem):
    sid = lax.axis_index("s")
    wid = sid * _NC + lax.axis_index("c")
    base = wid * _B_PER_W
    r0 = jnp.minimum(sid * _ROWS_PER_TILE, _ROWS - _ROWS_PER_TILE)
    pltpu.sync_copy(table_hbm.at[pl.ds(r0, _ROWS_PER_TILE)],
                    table_sh.at[pl.ds(r0, _ROWS_PER_TILE)])

    pltpu.sync_copy(idx_hbm.at[pl.ds(base, _B_PER_W)], idx_v)
    plsc.subcore_barrier()
    wrs = []
    for c in range(_NCH):
        lo = c * _CHUNK
        pltpu.async_copy(table_sh.at[idx_v.at[pl.ds(lo, _CHUNK)]],
                         rows_v.at[pl.ds(lo, _CHUNK)], gsem).wait()
        wrs.append(pltpu.async_copy(rows_v.at[pl.ds(lo, _CHUNK)],
                                    out_hbm.at[pl.ds(base + lo, _CHUNK)], wsem))
    for w in wrs:
        w.wait()


def kernel(t, embedding_weight):
    return _gather_kernel(t.astype(jnp.int32), embedding_weight)
